# Initial kernel scaffold; baseline (speedup 1.0000x reference)
import jax
import jax.numpy as jnp
from jax.experimental import pallas as pl


def kernel(x, edge_index, W1a, b1a, W1b, b1b, W2a, b2a, W2b, b2b):
    raise NotImplementedError("write your pallas kernel here")


# speedup vs baseline: 4.6808x; 4.6808x over previous
import functools

import jax
import jax.numpy as jnp
from jax import lax
from jax.experimental import pallas as pl
from jax.experimental.pallas import tpu as pltpu
from jax.experimental.pallas import tpu_sc as plsc

_NC = 2
_NS = 16
_CHUNK = 80


def _segment_sum_partials(x, src, dst):
    n, d = x.shape
    e = src.shape[0]
    nw = _NC * _NS
    epw = e // nw
    nchunk = epw // _CHUNK
    rpt = (n // _NS) // 8 * 8
    tail0 = _NS * rpt
    ntail = n - tail0

    mesh = plsc.VectorSubcoreMesh(core_axis_name="c", subcore_axis_name="s")

    @functools.partial(
        pl.kernel,
        out_type=(
            jax.ShapeDtypeStruct((n, d), jnp.float32),
            jax.ShapeDtypeStruct((n, d), jnp.float32),
        ),
        mesh=mesh,
        scratch_types=[
            pltpu.VMEM((_CHUNK,), jnp.int32),
            pltpu.VMEM((_CHUNK,), jnp.int32),
            pltpu.VMEM((_CHUNK, d), jnp.float32),
            pltpu.VMEM_SHARED((n, d), jnp.float32),
        ],
    )
    def sc_kernel(x_hbm, src_hbm, dst_hbm, p0_hbm, p1_hbm,
                  src_v, dst_v, rows_v, acc):
        c = lax.axis_index("c")
        s = lax.axis_index("s")
        wid = c * _NS + s
        row0 = pl.multiple_of(s * rpt, 8)

        pltpu.sync_copy(x_hbm.at[pl.ds(row0, rpt)], acc.at[pl.ds(row0, rpt)])
        if ntail:
            @pl.when(s == 0)
            def _():
                pltpu.sync_copy(x_hbm.at[pl.ds(tail0, ntail)],
                                acc.at[pl.ds(tail0, ntail)])
        plsc.subcore_barrier()

        @pl.loop(0, nchunk)
        def _(i):
            base = pl.multiple_of(wid * epw + i * _CHUNK, 8)
            pltpu.sync_copy(src_hbm.at[pl.ds(base, _CHUNK)], src_v)
            pltpu.sync_copy(dst_hbm.at[pl.ds(base, _CHUNK)], dst_v)
            pltpu.sync_copy(x_hbm.at[src_v], rows_v)
            pltpu.sync_copy(rows_v, acc.at[dst_v], add=True)

        plsc.subcore_barrier()

        @pl.when(c == 0)
        def _():
            pltpu.sync_copy(acc.at[pl.ds(row0, rpt)],
                            p0_hbm.at[pl.ds(row0, rpt)])
            if ntail:
                @pl.when(s == 0)
                def _():
                    pltpu.sync_copy(acc.at[pl.ds(tail0, ntail)],
                                    p0_hbm.at[pl.ds(tail0, ntail)])

        @pl.when(c == 1)
        def _():
            pltpu.sync_copy(acc.at[pl.ds(row0, rpt)],
                            p1_hbm.at[pl.ds(row0, rpt)])
            if ntail:
                @pl.when(s == 0)
                def _():
                    pltpu.sync_copy(acc.at[pl.ds(tail0, ntail)],
                                    p1_hbm.at[pl.ds(tail0, ntail)])

    return sc_kernel(x, src, dst)


def _mlp(p0, p1, xin, wa, ba, wb, bb, relu_out):
    n, d = xin.shape
    o = wb.shape[1]
    br = 1000

    def body(p0_ref, p1_ref, x_ref, wa_ref, ba_ref, wb_ref, bb_ref, o_ref):
        hin = p0_ref[...] + p1_ref[...] - x_ref[...]
        h = jnp.dot(hin, wa_ref[...], preferred_element_type=jnp.float32)
        h = jnp.maximum(h + ba_ref[...], 0.0)
        h = jnp.dot(h, wb_ref[...], preferred_element_type=jnp.float32)
        h = h + bb_ref[...]
        if relu_out:
            h = jnp.maximum(h, 0.0)
        o_ref[...] = h

    return pl.pallas_call(
        body,
        grid=(n // br,),
        in_specs=[
            pl.BlockSpec((br, d), lambda i: (i, 0)),
            pl.BlockSpec((br, d), lambda i: (i, 0)),
            pl.BlockSpec((br, d), lambda i: (i, 0)),
            pl.BlockSpec((d, wa.shape[1]), lambda i: (0, 0)),
            pl.BlockSpec((1, wa.shape[1]), lambda i: (0, 0)),
            pl.BlockSpec((wb.shape[0], o), lambda i: (0, 0)),
            pl.BlockSpec((1, o), lambda i: (0, 0)),
        ],
        out_specs=pl.BlockSpec((br, o), lambda i: (i, 0)),
        out_shape=jax.ShapeDtypeStruct((n, o), jnp.float32),
    )(p0, p1, xin, wa, ba.reshape(1, -1), wb, bb.reshape(1, -1))


def kernel(x, edge_index, W1a, b1a, W1b, b1b, W2a, b2a, W2b, b2b):
    src = edge_index[0].astype(jnp.int32)
    dst = edge_index[1].astype(jnp.int32)

    p0, p1 = _segment_sum_partials(x, src, dst)
    h1 = _mlp(p0, p1, x, W1a, b1a, W1b, b1b, relu_out=True)

    q0, q1 = _segment_sum_partials(h1, src, dst)
    out = _mlp(q0, q1, h1, W2a, b2a, W2b, b2b, relu_out=False)
    return out


# speedup vs baseline: 12.6026x; 2.6924x over previous
import functools

import jax
import jax.numpy as jnp
from jax import lax
from jax.experimental import pallas as pl
from jax.experimental.pallas import tpu as pltpu
from jax.experimental.pallas import tpu_sc as plsc

_NC = 2
_NS = 16
_CHUNK = 80


_NROW = 3
_NIDX = 6


def _segment_sum_partials(x, edges3):
    n, d = x.shape
    nchunk = edges3.shape[0] // (_NC * _NS)
    rpt = (n // _NS) // 8 * 8
    tail0 = _NS * rpt
    ntail = n - tail0

    mesh = plsc.VectorSubcoreMesh(core_axis_name="c", subcore_axis_name="s")

    @functools.partial(
        pl.kernel,
        out_type=(
            jax.ShapeDtypeStruct((n, d), jnp.float32),
            jax.ShapeDtypeStruct((n, d), jnp.float32),
        ),
        mesh=mesh,
        scratch_types=[
            pltpu.VMEM((_NIDX, 2, _CHUNK), jnp.int32),
            pltpu.VMEM((_NROW, _CHUNK, d), jnp.float32),
            pltpu.VMEM_SHARED((n, d), jnp.float32),
            pltpu.SemaphoreType.DMA,
            pltpu.SemaphoreType.DMA,
            pltpu.SemaphoreType.DMA,
            pltpu.SemaphoreType.DMA,
            pltpu.SemaphoreType.DMA,
            pltpu.SemaphoreType.DMA,
            pltpu.SemaphoreType.DMA,
            pltpu.SemaphoreType.DMA,
            pltpu.SemaphoreType.DMA,
            pltpu.SemaphoreType.DMA,
        ],
    )
    def sc_kernel(x_hbm, e_hbm, p0_hbm, p1_hbm,
                  idx_v, rows_v, acc,
                  g0, g1, g2, x0, x1, x2, x3, x4, x5, isem):
        gsems = (g0, g1, g2)
        xsems = (x0, x1, x2, x3, x4, x5)
        c = lax.axis_index("c")
        s = lax.axis_index("s")
        wid = c * _NS + s
        chunk0 = wid * nchunk
        row0 = pl.multiple_of(s * rpt, 8)

        def fire_idx(j, b6):
            pltpu.async_copy(e_hbm.at[chunk0 + j], idx_v.at[b6], xsems[b6])

        def fire_gather(j, b3, b6):
            pltpu.make_async_copy(
                e_hbm.at[chunk0 + j], idx_v.at[b6], xsems[b6]).wait()
            pltpu.async_copy(x_hbm.at[idx_v.at[b6].at[0]],
                             rows_v.at[b3], gsems[b3])

        init_cp = pltpu.async_copy(
            x_hbm.at[pl.ds(row0, rpt)], acc.at[pl.ds(row0, rpt)], isem)
        for j in range(_NIDX):
            fire_idx(j, j)
        for j in range(_NROW):
            fire_gather(j, j, j)
        init_cp.wait()
        if ntail:
            @pl.when(s == 0)
            def _():
                pltpu.sync_copy(x_hbm.at[pl.ds(tail0, ntail)],
                                acc.at[pl.ds(tail0, ntail)])
        plsc.subcore_barrier()

        def body(j, b3, b6, b6n, static):
            pltpu.make_async_copy(
                x_hbm.at[idx_v.at[b6].at[0]], rows_v.at[b3],
                gsems[b3]).wait()
            pltpu.sync_copy(rows_v.at[b3], acc.at[idx_v.at[b6].at[1]],
                            add=True)
            if static:
                if j + _NIDX < nchunk:
                    fire_idx(j + _NIDX, b6)
                if j + _NROW < nchunk:
                    fire_gather(j + _NROW, b3, b6n)
            else:
                @pl.when(j + _NIDX < nchunk)
                def _():
                    fire_idx(j + _NIDX, b6)

                @pl.when(j + _NROW < nchunk)
                def _():
                    fire_gather(j + _NROW, b3, b6n)

        main = nchunk - nchunk % _NIDX

        @pl.loop(0, main, step=_NIDX)
        def _(g):
            for u in range(_NIDX):
                body(g + u, u % _NROW, u, (u + _NROW) % _NIDX, static=False)

        for j in range(main, nchunk):
            body(j, j % _NROW, j % _NIDX, (j + _NROW) % _NIDX, static=True)

        plsc.subcore_barrier()

        @pl.when(c == 0)
        def _():
            pltpu.sync_copy(acc.at[pl.ds(row0, rpt)],
                            p0_hbm.at[pl.ds(row0, rpt)])
            if ntail:
                @pl.when(s == 0)
                def _():
                    pltpu.sync_copy(acc.at[pl.ds(tail0, ntail)],
                                    p0_hbm.at[pl.ds(tail0, ntail)])

        @pl.when(c == 1)
        def _():
            pltpu.sync_copy(acc.at[pl.ds(row0, rpt)],
                            p1_hbm.at[pl.ds(row0, rpt)])
            if ntail:
                @pl.when(s == 0)
                def _():
                    pltpu.sync_copy(acc.at[pl.ds(tail0, ntail)],
                                    p1_hbm.at[pl.ds(tail0, ntail)])

    return sc_kernel(x, edges3)


def _mlp(p0, p1, xin, wa, ba, wb, bb, relu_out):
    n, d = xin.shape
    o = wb.shape[1]
    br = 1000

    def body(p0_ref, p1_ref, x_ref, wa_ref, ba_ref, wb_ref, bb_ref, o_ref):
        hin = p0_ref[...] + p1_ref[...] - x_ref[...]
        h = jnp.dot(hin, wa_ref[...], preferred_element_type=jnp.float32)
        h = jnp.maximum(h + ba_ref[...], 0.0)
        h = jnp.dot(h, wb_ref[...], preferred_element_type=jnp.float32)
        h = h + bb_ref[...]
        if relu_out:
            h = jnp.maximum(h, 0.0)
        o_ref[...] = h

    return pl.pallas_call(
        body,
        grid=(n // br,),
        in_specs=[
            pl.BlockSpec((br, d), lambda i: (i, 0)),
            pl.BlockSpec((br, d), lambda i: (i, 0)),
            pl.BlockSpec((br, d), lambda i: (i, 0)),
            pl.BlockSpec((d, wa.shape[1]), lambda i: (0, 0)),
            pl.BlockSpec((1, wa.shape[1]), lambda i: (0, 0)),
            pl.BlockSpec((wb.shape[0], o), lambda i: (0, 0)),
            pl.BlockSpec((1, o), lambda i: (0, 0)),
        ],
        out_specs=pl.BlockSpec((br, o), lambda i: (i, 0)),
        out_shape=jax.ShapeDtypeStruct((n, o), jnp.float32),
    )(p0, p1, xin, wa, ba.reshape(1, -1), wb, bb.reshape(1, -1))


def kernel(x, edge_index, W1a, b1a, W1b, b1b, W2a, b2a, W2b, b2b):
    e = edge_index.shape[1]
    nck = e // _CHUNK
    edges3 = jnp.stack(
        [edge_index[0].astype(jnp.int32).reshape(nck, _CHUNK),
         edge_index[1].astype(jnp.int32).reshape(nck, _CHUNK)], axis=1)

    p0, p1 = _segment_sum_partials(x, edges3)
    h1 = _mlp(p0, p1, x, W1a, b1a, W1b, b1b, relu_out=True)

    q0, q1 = _segment_sum_partials(h1, edges3)
    out = _mlp(q0, q1, h1, W2a, b2a, W2b, b2b, relu_out=False)
    return out
